# baseline (speedup 1.0000x reference)
import functools

import jax
import jax.numpy as jnp
from jax import lax
from jax.experimental import pallas as pl
from jax.experimental.pallas import tpu as pltpu
from jax.experimental.pallas import tpu_sc as plsc

IN_DIM = 32768
OUT_DIM = 10
K = 1024
BATCH = 128
INT_MIN = -2147483648

NT = 16
NC = 2
NW = NT * NC
ROWS_W = BATCH // NW
CHUNK = IN_DIM // NT
NV = CHUNK // 16
UNROLL = 8


def _unrolled(n, body, init):
    def outer(i, carry):
        for u in range(UNROLL):
            carry = body(i * UNROLL + u, carry)
        return carry

    return lax.fori_loop(0, n // UNROLL, outer, init)


def _splat(v, lane, l):
    return jnp.broadcast_to(jnp.sum(jnp.where(lane == l, v, 0)), (16,))


def _sc_body(logits_hbm, xflat_hbm, w1_hbm, mask_hbm, xg_hbm, w1g_hbm,
             lv, keys, hist, hred, g16, teq, mk, idxl, idxall, posb, xgb,
             sp_hist, sp_g, sp_idx, sp_cnt, sem):
    c = lax.axis_index("c")
    t = lax.axis_index("s")
    wid = t * NC + c
    base = t * CHUNK
    lane = lax.iota(jnp.int32, 16)
    ones16 = jnp.ones((16,), jnp.int32)
    zeros16 = jnp.zeros((16,), jnp.int32)

    pltpu.sync_copy(logits_hbm.at[pl.ds(base, CHUNK)], lv)

    pref = zeros16
    rem = jnp.full((16,), K, jnp.int32)
    for p in range(4):
        sh = 24 - 8 * p

        def zero(i, carry):
            hist[pl.ds(i * 16, 16)] = zeros16
            return carry

        _unrolled(256, zero, 0)

        pref_u = pref.astype(jnp.uint32)

        def scan(v, carry):
            if p == 0:
                x16 = lv[pl.ds(v * 16, 16)]
                b = lax.bitcast_convert_type(x16, jnp.int32)
                k = b ^ (lax.shift_right_arithmetic(
                    b, jnp.full((16,), 31, jnp.int32))
                    & jnp.int32(0x7FFFFFFF))
                k = jnp.where(b == jnp.int32(INT_MIN), jnp.int32(0), k)
                keys[pl.ds(v * 16, 16)] = k
            else:
                k = keys[pl.ds(v * 16, 16)]
            uk = lax.bitcast_convert_type(k, jnp.uint32) ^ jnp.uint32(0x80000000)
            bucket = ((uk >> jnp.uint32(sh)) & jnp.uint32(0xFF)).astype(jnp.int32)
            idx = lane * 256 + bucket
            if p == 0:
                plsc.addupdate_scatter(hist, [idx], ones16)
            else:
                act = (uk >> jnp.uint32(32 - 8 * p)) == pref_u
                plsc.addupdate_scatter(hist, [idx], ones16, mask=act)
            return carry

        _unrolled(NV, scan, 0)

        def red_c(cc, carry):
            acc = zeros16
            for l in range(16):
                acc = acc + hist[pl.ds(l * 256 + cc * 16, 16)]
            hred[pl.ds(cc * 16, 16)] = acc
            return carry

        lax.fori_loop(0, 16, red_c, 0)

        copies = [pltpu.make_async_copy(
            hred.at[pl.ds(g * 16, 16)],
            sp_hist.at[pl.ds(((p * 16 + g) * 16 + t) * 16, 16)],
            sem) for g in range(16)]
        for cp in copies:
            cp.start()
        for cp in copies:
            cp.wait()
        plsc.subcore_barrier()

        pltpu.sync_copy(sp_hist.at[pl.ds((p * 16 + t) * 256, 256)], teq)
        acc = zeros16
        for l in range(16):
            acc = acc + teq[pl.ds(l * 16, 16)]
        g16[...] = acc
        pltpu.sync_copy(g16, sp_g.at[pl.ds(p * 256 + t * 16, 16)])
        plsc.subcore_barrier()

        pltpu.sync_copy(sp_g.at[pl.ds(p * 256, 256)], hred)

        def select(j, carry):
            found, bsel, rem2, cum = carry
            jd = 15 - j
            acc16 = hred[pl.ds(jd * 16, 16)]
            F = jnp.flip(plsc.cumsum(jnp.flip(acc16, 0)), 0) + cum
            m = F >= rem
            pop = plsc.all_reduce_population_count(m)
            l = pop - 1
            fsel = _splat(F, lane, l)
            asel = _splat(acc16, lane, l)
            qual = jnp.logical_and(found == 0, pop > 0)
            found = jnp.where(qual, 1, found)
            bsel = jnp.where(qual, jd * 16 + l, bsel)
            rem2 = jnp.where(qual, rem - (fsel - asel), rem2)
            cum = _splat(F, lane, zeros16)
            return found, bsel, rem2, cum

        _, bsel, rem, _ = lax.fori_loop(
            0, 16, select, (zeros16, zeros16, rem, zeros16))
        pref = (pref << 8) | bsel

    thr = pref ^ jnp.int32(INT_MIN)

    bg = jnp.sum(jnp.where(lane == 0, bsel >> 4, 0))
    pltpu.sync_copy(sp_hist.at[pl.ds((3 * 16 + bg) * 256, 256)], teq)
    w = plsc.load_gather(teq, [lane * 16 + (bsel & 15)])
    excl = plsc.cumsum(w) - w
    mybase = jnp.sum(jnp.where(lane == t, excl, 0))
    myeq = jnp.sum(jnp.where(lane == t, w, 0))
    take = jnp.clip(rem - mybase, 0, myeq)

    def mkwrite(ties):
        def go(v, carry):
            cnt, off = carry
            k = keys[pl.ds(v * 16, 16)]
            gt = k > thr
            if ties:
                eq = k == thr
                r16 = plsc.cumsum(eq.astype(jnp.int32)) + cnt
                sel = gt | (eq & (r16 <= take))
                cnt = cnt + jnp.sum(eq.astype(jnp.int32))
            else:
                sel = gt
            mk[pl.ds(v * 16, 16)] = jnp.where(sel, 1.0, 0.0)
            ranks = plsc.cumsum(sel.astype(jnp.int32))
            d = off + ranks - 1
            plsc.store_scatter(idxl, [d], base + v * 16 + lane, mask=sel)
            off = off + plsc.all_reduce_population_count(sel)
            return cnt, off

        return go

    def write_fast(_):
        return lax.fori_loop(0, NV, mkwrite(False), (jnp.int32(0), zeros16))

    def write_ties(_):
        return lax.fori_loop(0, NV, mkwrite(True), (jnp.int32(0), zeros16))

    _, off = lax.cond(jnp.any(take > 0), write_ties, write_fast, 0)
    nsel = jnp.sum(jnp.where(lane == 0, off, 0))

    @pl.when(c == 0)
    def _():
        pltpu.sync_copy(mk, mask_hbm.at[pl.ds(base, CHUNK)])

    g16[...] = jnp.broadcast_to(nsel, (16,))
    pltpu.sync_copy(g16, sp_cnt.at[pl.ds(t * 16, 16)])

    def pub(cc, carry):
        pltpu.sync_copy(idxl.at[pl.ds(cc * 128, 128)],
                        sp_idx.at[pl.ds(t * 2048 + cc * 128, 128)])
        return carry

    lax.fori_loop(0, (nsel + 127) // 128, pub, 0)
    plsc.subcore_barrier()

    pltpu.sync_copy(sp_cnt, teq)
    wcnt = plsc.load_gather(teq, [lane * 16 + lane])
    woff = plsc.cumsum(wcnt) - wcnt

    def merge(tt, carry):
        cnt_s = jnp.sum(jnp.where(lane == tt, wcnt, 0))
        off_s = jnp.sum(jnp.where(lane == tt, woff, 0))
        off_v = jnp.broadcast_to(off_s, (16,))

        def cpy(cc, carry2):
            pltpu.sync_copy(sp_idx.at[pl.ds(tt * 2048 + cc * 128, 128)],
                            idxl.at[pl.ds(cc * 128, 128)])
            return carry2

        lax.fori_loop(0, (cnt_s + 127) // 128, cpy, 0)

        def scat(g, carry2):
            vals = idxl[pl.ds(g * 16, 16)]
            valid = (g * 16 + lane) < cnt_s
            d = off_v + g * 16 + lane
            plsc.store_scatter(idxall, [d], vals, mask=valid)
            return carry2

        lax.fori_loop(0, (cnt_s + 15) // 16, scat, 0)
        return carry

    lax.fori_loop(0, 16, merge, 0)

    for r in range(ROWS_W):
        b_id = wid * ROWS_W + r

        def pos(g, carry, _r=r, _b=b_id):
            posb[pl.ds(_r * K + g * 16, 16)] = (
                idxall[pl.ds(g * 16, 16)] + _b * IN_DIM)
            return carry

        lax.fori_loop(0, K // 16, pos, 0)

    gathers = [pltpu.make_async_copy(
        xflat_hbm.at[posb.at[pl.ds(cc * 128, 128)]],
        xgb.at[pl.ds(cc * 128, 128)],
        sem) for cc in range(ROWS_W * K // 128)]
    for cp in gathers:
        cp.start()
    for cp in gathers:
        cp.wait()
    pltpu.sync_copy(xgb, xg_hbm.at[pl.ds(wid * ROWS_W * K, ROWS_W * K)])

    @pl.when(wid < 8)
    def _():
        for cc in range(4):
            def wpos(g, carry, _cc=cc):
                col = wid * 4 + _cc
                posb[pl.ds(_cc * K + g * 16, 16)] = (
                    idxall[pl.ds(g * 16, 16)] * 32 + col)
                return carry

            lax.fori_loop(0, K // 16, wpos, 0)

        wgathers = [pltpu.make_async_copy(
            w1_hbm.at[posb.at[pl.ds(cc * 128, 128)]],
            xgb.at[pl.ds(cc * 128, 128)],
            sem) for cc in range(4 * K // 128)]
        for cp in wgathers:
            cp.start()
        for cp in wgathers:
            cp.wait()
        pltpu.sync_copy(xgb, w1g_hbm.at[pl.ds(wid * 4 * K, 4 * K)])


@functools.lru_cache(maxsize=1)
def _sc_kernel():
    mesh = plsc.VectorSubcoreMesh(core_axis_name="c", subcore_axis_name="s")
    return pl.kernel(
        _sc_body,
        out_type=(
            jax.ShapeDtypeStruct((IN_DIM,), jnp.float32),
            jax.ShapeDtypeStruct((BATCH * K,), jnp.float32),
            jax.ShapeDtypeStruct((32 * K,), jnp.float32),
        ),
        mesh=mesh,
        compiler_params=pltpu.CompilerParams(needs_layout_passes=False),
        scratch_types=[
            pltpu.VMEM((CHUNK,), jnp.float32),
            pltpu.VMEM((CHUNK,), jnp.int32),
            pltpu.VMEM((4096,), jnp.int32),
            pltpu.VMEM((256,), jnp.int32),
            pltpu.VMEM((16,), jnp.int32),
            pltpu.VMEM((256,), jnp.int32),
            pltpu.VMEM((CHUNK,), jnp.float32),
            pltpu.VMEM((CHUNK + 16,), jnp.int32),
            pltpu.VMEM((K + 16,), jnp.int32),
            pltpu.VMEM((ROWS_W * K,), jnp.int32),
            pltpu.VMEM((ROWS_W * K,), jnp.float32),
            pltpu.VMEM_SHARED((16384,), jnp.int32),
            pltpu.VMEM_SHARED((1024,), jnp.int32),
            pltpu.VMEM_SHARED((NT * 2048,), jnp.int32),
            pltpu.VMEM_SHARED((256,), jnp.int32),
            pltpu.SemaphoreType.DMA,
        ],
    )


def _mlp_body(xg_ref, w1gt_ref, b1_ref, w2_ref, b2_ref, w3_ref, b3_ref,
              out_ref):
    h1 = lax.dot_general(
        xg_ref[...], w1gt_ref[...], (((1,), (1,)), ((), ())),
        preferred_element_type=jnp.float32)
    h = jnp.maximum(h1 + b1_ref[...], 0.0)
    h = jnp.maximum(
        jnp.dot(h, w2_ref[...], preferred_element_type=jnp.float32)
        + b2_ref[...], 0.0)
    out_ref[...] = (
        jnp.dot(h, w3_ref[...], preferred_element_type=jnp.float32)
        + b3_ref[...])


@jax.jit
def kernel(x, logits, W1, b1, W2, b2, W3, b3, epoch, total_epochs, training):
    del epoch, total_epochs, training
    mask, xg, w1gt = _sc_kernel()(logits, x.reshape(-1), W1.reshape(-1))

    out = pl.pallas_call(
        _mlp_body,
        out_shape=jax.ShapeDtypeStruct((BATCH, OUT_DIM), jnp.float32),
    )(xg.reshape(BATCH, K), w1gt.reshape(32, K), b1.reshape(1, 32),
      W2, b2.reshape(1, 16), W3, b3.reshape(1, OUT_DIM))

    return out, mask


# speedup vs baseline: 1.8661x; 1.8661x over previous
import functools

import jax
import jax.numpy as jnp
from jax import lax
from jax.experimental import pallas as pl
from jax.experimental.pallas import tpu as pltpu
from jax.experimental.pallas import tpu_sc as plsc

IN_DIM = 32768
OUT_DIM = 10
K = 1024
BATCH = 128
BLK = 8192
N_BLK = IN_DIM // BLK
INT_MIN = -2147483648

NT = 16
CHUNK = IN_DIM // NT
NV = CHUNK // 16
UNROLL = 8


def _unrolled(n, body, init):
    def outer(i, carry):
        for u in range(UNROLL):
            carry = body(i * UNROLL + u, carry)
        return carry

    return lax.fori_loop(0, n // UNROLL, outer, init)


def _splat(v, lane, l):
    return jnp.broadcast_to(jnp.sum(jnp.where(lane == l, v, 0)), (16,))


def _sc_mask_body(logits_hbm, mask_hbm,
                  lv, keys, hist, hred, g16, teq, mk, sp_hist, sp_g, sem):
    t = lax.axis_index("s")
    base = t * CHUNK
    lane = lax.iota(jnp.int32, 16)
    ones16 = jnp.ones((16,), jnp.int32)
    zeros16 = jnp.zeros((16,), jnp.int32)

    pltpu.sync_copy(logits_hbm.at[pl.ds(base, CHUNK)], lv)

    pref = zeros16
    rem = jnp.full((16,), K, jnp.int32)
    for p in range(4):
        sh = 24 - 8 * p

        def zero(i, carry):
            hist[pl.ds(i * 16, 16)] = zeros16
            return carry

        _unrolled(256, zero, 0)

        pref_u = pref.astype(jnp.uint32)

        def scan(v, carry):
            if p == 0:
                x16 = lv[pl.ds(v * 16, 16)]
                b = lax.bitcast_convert_type(x16, jnp.int32)
                k = b ^ (lax.shift_right_arithmetic(
                    b, jnp.full((16,), 31, jnp.int32))
                    & jnp.int32(0x7FFFFFFF))
                k = jnp.where(b == jnp.int32(INT_MIN), jnp.int32(0), k)
                keys[pl.ds(v * 16, 16)] = k
            else:
                k = keys[pl.ds(v * 16, 16)]
            uk = lax.bitcast_convert_type(k, jnp.uint32) ^ jnp.uint32(0x80000000)
            bucket = ((uk >> jnp.uint32(sh)) & jnp.uint32(0xFF)).astype(jnp.int32)
            idx = lane * 256 + bucket
            if p == 0:
                plsc.addupdate_scatter(hist, [idx], ones16)
            else:
                act = (uk >> jnp.uint32(32 - 8 * p)) == pref_u.astype(jnp.uint32)
                plsc.addupdate_scatter(hist, [idx], ones16, mask=act)
            return carry

        _unrolled(NV, scan, 0)

        def red_c(cc, carry):
            acc = zeros16
            for l in range(16):
                acc = acc + hist[pl.ds(l * 256 + cc * 16, 16)]
            hred[pl.ds(cc * 16, 16)] = acc
            return carry

        lax.fori_loop(0, 16, red_c, 0)

        copies = []
        for g in range(16):
            copies.append(pltpu.make_async_copy(
                hred.at[pl.ds(g * 16, 16)],
                sp_hist.at[pl.ds(((p * 16 + g) * 16 + t) * 16, 16)],
                sem))
        for cp in copies:
            cp.start()
        for cp in copies:
            cp.wait()
        plsc.subcore_barrier()

        pltpu.sync_copy(sp_hist.at[pl.ds((p * 16 + t) * 256, 256)], teq)
        acc = zeros16
        for l in range(16):
            acc = acc + teq[pl.ds(l * 16, 16)]
        g16[...] = acc
        pltpu.sync_copy(g16, sp_g.at[pl.ds(p * 256 + t * 16, 16)])
        plsc.subcore_barrier()

        pltpu.sync_copy(sp_g.at[pl.ds(p * 256, 256)], hred)

        def select(j, carry):
            found, bsel, rem2, cum = carry
            jd = 15 - j
            acc16 = hred[pl.ds(jd * 16, 16)]
            F = jnp.flip(plsc.cumsum(jnp.flip(acc16, 0)), 0) + cum
            m = F >= rem
            pop = plsc.all_reduce_population_count(m)
            l = pop - 1
            fsel = _splat(F, lane, l)
            asel = _splat(acc16, lane, l)
            qual = jnp.logical_and(found == 0, pop > 0)
            found = jnp.where(qual, 1, found)
            bsel = jnp.where(qual, jd * 16 + l, bsel)
            rem2 = jnp.where(qual, rem - (fsel - asel), rem2)
            cum = _splat(F, lane, zeros16)
            return found, bsel, rem2, cum

        _, bsel, rem, _ = lax.fori_loop(
            0, 16, select, (zeros16, zeros16, rem, zeros16))
        pref = (pref << 8) | bsel

    thr = pref ^ jnp.int32(INT_MIN)

    bg = jnp.sum(jnp.where(lane == 0, bsel >> 4, 0))
    pltpu.sync_copy(sp_hist.at[pl.ds((3 * 16 + bg) * 256, 256)], teq)
    w = plsc.load_gather(teq, [lane * 16 + (bsel & 15)])
    excl = plsc.cumsum(w) - w
    mybase = jnp.sum(jnp.where(lane == t, excl, 0))
    myeq = jnp.sum(jnp.where(lane == t, w, 0))
    take = jnp.clip(rem - mybase, 0, myeq)

    def write_fast(_):
        def go(v, carry):
            k = keys[pl.ds(v * 16, 16)]
            mk[pl.ds(v * 16, 16)] = jnp.where(k > thr, 1.0, 0.0)
            return carry

        return _unrolled(NV, go, jnp.int32(0))

    def write_ties(_):
        def go(v, cnt):
            k = keys[pl.ds(v * 16, 16)]
            gt = k > thr
            eq = k == thr
            r16 = plsc.cumsum(eq.astype(jnp.int32)) + cnt
            sel = gt | (eq & (r16 <= take))
            mk[pl.ds(v * 16, 16)] = jnp.where(sel, 1.0, 0.0)
            return cnt + jnp.sum(eq.astype(jnp.int32))

        return lax.fori_loop(0, NV, go, jnp.int32(0))

    lax.cond(jnp.any(take > 0), write_ties, write_fast, 0)

    pltpu.sync_copy(mk, mask_hbm.at[pl.ds(base, CHUNK)])


@functools.lru_cache(maxsize=1)
def _sc_mask_kernel():
    mesh = plsc.VectorSubcoreMesh(
        core_axis_name="c", subcore_axis_name="s", num_cores=1)
    return pl.kernel(
        _sc_mask_body,
        out_type=jax.ShapeDtypeStruct((IN_DIM,), jnp.float32),
        mesh=mesh,
        compiler_params=pltpu.CompilerParams(needs_layout_passes=False),
        scratch_types=[
            pltpu.VMEM((CHUNK,), jnp.float32),
            pltpu.VMEM((CHUNK,), jnp.int32),
            pltpu.VMEM((4096,), jnp.int32),
            pltpu.VMEM((256,), jnp.int32),
            pltpu.VMEM((16,), jnp.int32),
            pltpu.VMEM((256,), jnp.int32),
            pltpu.VMEM((CHUNK,), jnp.float32),
            pltpu.VMEM_SHARED((16384,), jnp.int32),
            pltpu.VMEM_SHARED((1024,), jnp.int32),
            pltpu.SemaphoreType.DMA,
        ],
    )


def _mlp_body(x_ref, m_ref, w1_ref, b1_ref, w2_ref, b2_ref, w3_ref, b3_ref,
              out_ref, acc_ref):
    i = pl.program_id(0)

    @pl.when(i == 0)
    def _():
        acc_ref[...] = jnp.zeros_like(acc_ref)

    xm = x_ref[...] * m_ref[...]
    acc_ref[...] += jnp.dot(xm, w1_ref[...], preferred_element_type=jnp.float32)

    @pl.when(i == N_BLK - 1)
    def _():
        h = jnp.maximum(acc_ref[...] + b1_ref[...], 0.0)
        h = jnp.maximum(
            jnp.dot(h, w2_ref[...], preferred_element_type=jnp.float32)
            + b2_ref[...], 0.0)
        out_ref[...] = (
            jnp.dot(h, w3_ref[...], preferred_element_type=jnp.float32)
            + b3_ref[...])


@jax.jit
def kernel(x, logits, W1, b1, W2, b2, W3, b3, epoch, total_epochs, training):
    del epoch, total_epochs, training
    mask = _sc_mask_kernel()(logits)
    mask2 = mask.reshape(1, IN_DIM)

    out = pl.pallas_call(
        _mlp_body,
        grid=(N_BLK,),
        in_specs=[
            pl.BlockSpec((BATCH, BLK), lambda i: (0, i)),
            pl.BlockSpec((1, BLK), lambda i: (0, i)),
            pl.BlockSpec((BLK, 32), lambda i: (i, 0)),
            pl.BlockSpec((1, 32), lambda i: (0, 0)),
            pl.BlockSpec((32, 16), lambda i: (0, 0)),
            pl.BlockSpec((1, 16), lambda i: (0, 0)),
            pl.BlockSpec((16, OUT_DIM), lambda i: (0, 0)),
            pl.BlockSpec((1, OUT_DIM), lambda i: (0, 0)),
        ],
        out_specs=pl.BlockSpec((BATCH, OUT_DIM), lambda i: (0, 0)),
        out_shape=jax.ShapeDtypeStruct((BATCH, OUT_DIM), jnp.float32),
        scratch_shapes=[pltpu.VMEM((BATCH, 32), jnp.float32)],
    )(x, mask2, W1, b1.reshape(1, 32), W2, b2.reshape(1, 16), W3,
      b3.reshape(1, OUT_DIM))

    return out, mask
